# baseline (speedup 1.0000x reference)
import jax
import jax.numpy as jnp
from jax import lax
from jax.experimental import pallas as pl
from jax.experimental.pallas import tpu as pltpu
from jax.experimental.pallas import tpu_sc as plsc

NUM_CLASSES = 1000
D = 128
K = 20
B = 1024
S = 4
N = B * S
LEVELS = 128
EMBED_SCALE = float(D) ** 0.5

NC = 2
NS = 16
NW = NC * NS
SEQ_PER_W = N // NW
CH = 2
NCHUNK = SEQ_PER_W // CH
ROWS = CH * K
NV = D // 16
NBUF = 4


def _tec_body(ci_hbm, vals_hbm, bp_hbm, wcat_hbm, pos_hbm,
              cp_hbm, vp_hbm, pos_v, ci_all, valx, vals_all, *ring):
    bps = ring[0:NBUF]
    cats = ring[NBUF:2 * NBUF]
    cpos = ring[2 * NBUF:3 * NBUF]
    vpos = ring[3 * NBUF:4 * NBUF]
    semgs = ring[4 * NBUF:5 * NBUF]
    semos = ring[5 * NBUF:6 * NBUF]

    wid = lax.axis_index("s") * NC + lax.axis_index("c")
    wbase = wid * SEQ_PER_W * K

    pltpu.sync_copy(pos_hbm, pos_v)
    pltpu.sync_copy(ci_hbm.at[pl.ds(wbase, SEQ_PER_W * K)], ci_all)
    pltpu.sync_copy(vals_hbm.at[pl.ds(wbase, SEQ_PER_W * K)],
                    vals_all.at[pl.ds(0, SEQ_PER_W * K)])

    def start(c, b):
        idx = ci_all.at[pl.ds(c * ROWS, ROWS)]
        pltpu.async_copy(
            bp_hbm.at[pl.ds(wid * SEQ_PER_W // S + c * CH // S, 1)],
            bps[b], semgs[b])
        pltpu.async_copy(wcat_hbm.at[idx], cats[b], semgs[b])

    def finish(c, b, j):
        bp, cat = bps[b], cats[b]
        cpo, vpo, semg, semo = cpos[b], vpos[b], semgs[b], semos[b]
        b_row = wid * SEQ_PER_W // S + c * CH // S
        s0 = (c % (S // CH)) * CH

        @pl.when(j >= 1)
        def _():
            pltpu.make_async_copy(cpo, cp_hbm.at[0, pl.ds(0, CH)], semo).wait()
            pltpu.make_async_copy(vpo, vp_hbm.at[0, pl.ds(0, CH)], semo).wait()

        for r in range(ROWS):
            g = vals_all[pl.ds(c * ROWS + (r // 16) * 16, 16)]
            valx[r, :] = lax.broadcast_in_dim(
                lax.slice_in_dim(g, r % 16, r % 16 + 1), (16,), (0,))

        pltpu.make_async_copy(bp_hbm.at[pl.ds(0, 1)], bp, semg).wait()
        pltpu.make_async_copy(wcat_hbm.at[pl.ds(0, ROWS)], cat, semg).wait()

        for s_local in range(CH):
            def dv_body(dv, _):
                sl = pl.ds(dv * 16, 16)
                acc = bp[0, sl]
                for k in range(K):
                    r = s_local * K + k
                    cp = acc + pos_v[k, sl]
                    cpo[s_local, k, sl] = cp
                    vpo[s_local, k, sl] = cp + cat[r, pl.ds(dv * 16 + 2 * D, 16)]
                    acc = (acc + cat[r, sl]
                           + valx[r, :] * cat[r, pl.ds(dv * 16 + D, 16)])
                return ()

            lax.fori_loop(0, NV, dv_body, (), unroll=False)

        pltpu.async_copy(cpo, cp_hbm.at[b_row, pl.ds(s0, CH)], semo)
        pltpu.async_copy(vpo, vp_hbm.at[b_row, pl.ds(s0, CH)], semo)

    for b in range(NBUF):
        start(b, b)

    def body(j, _):
        for b in range(NBUF):
            c = j * NBUF + b
            finish(c, b, j)

            @pl.when(c + NBUF < NCHUNK)
            def _():
                start(c + NBUF, b)
        return ()

    lax.fori_loop(0, NCHUNK // NBUF, body, (), unroll=False)

    for b in range(NBUF):
        pltpu.make_async_copy(cpos[b], cp_hbm.at[0, pl.ds(0, CH)],
                              semos[b]).wait()
        pltpu.make_async_copy(vpos[b], vp_hbm.at[0, pl.ds(0, CH)],
                              semos[b]).wait()


@jax.jit
def _predictor_sc(ci_flat, vals_flat, base_predictor, wcat, pos_s):
    mesh = plsc.VectorSubcoreMesh(core_axis_name="c", subcore_axis_name="s")
    scratch = [
        pltpu.VMEM((K, D), jnp.float32),
        pltpu.VMEM((SEQ_PER_W * K,), jnp.int32),
        pltpu.VMEM((ROWS, 16), jnp.float32),
        pltpu.VMEM((SEQ_PER_W * K + 16,), jnp.float32),
    ]
    scratch += [pltpu.VMEM((1, D), jnp.float32) for _ in range(NBUF)]
    scratch += [pltpu.VMEM((ROWS, 3 * D), jnp.float32) for _ in range(NBUF)]
    scratch += [pltpu.VMEM((CH, K, D), jnp.float32) for _ in range(NBUF)]
    scratch += [pltpu.VMEM((CH, K, D), jnp.float32) for _ in range(NBUF)]
    scratch += [pltpu.SemaphoreType.DMA for _ in range(NBUF)]
    scratch += [pltpu.SemaphoreType.DMA for _ in range(NBUF)]
    f = pl.kernel(
        _tec_body,
        out_type=(
            jax.ShapeDtypeStruct((B, S, K, D), jnp.float32),
            jax.ShapeDtypeStruct((B, S, K, D), jnp.float32),
        ),
        mesh=mesh,
        scratch_types=scratch,
    )
    return f(ci_flat, vals_flat, base_predictor, wcat, pos_s)


def kernel(class_indexes, value_indexes, base_predictor, W_present, W_value,
           W_query, position_embed):
    ci_flat = class_indexes.reshape(N * K)
    vals_flat = (value_indexes.astype(jnp.float32)
                 * (1.0 / (LEVELS - 1))).reshape(N * K)
    wcat = jnp.concatenate(
        [W_present, W_value, W_present + W_query], axis=1) * EMBED_SCALE
    pos_s = position_embed * EMBED_SCALE
    return _predictor_sc(ci_flat, vals_flat, base_predictor, wcat, pos_s)


# speedup vs baseline: 1.0361x; 1.0361x over previous
import jax
import jax.numpy as jnp
from jax import lax
from jax.experimental import pallas as pl
from jax.experimental.pallas import tpu as pltpu
from jax.experimental.pallas import tpu_sc as plsc

NUM_CLASSES = 1000
D = 128
K = 20
B = 1024
S = 4
N = B * S
LEVELS = 128
EMBED_SCALE = float(D) ** 0.5

NC = 2
NS = 16
NW = NC * NS
SEQ_PER_W = N // NW
CH = 2
NCHUNK = SEQ_PER_W // CH
ROWS = CH * K
NV = D // 16
NBUF = 4
VPAD = 48


def _tec_body(ci_hbm, vals_hbm, bp_hbm, wcat_hbm, pos_hbm,
              cp_hbm, vp_hbm, pos_v, ci_all, valx, *ring):
    bps = ring[0:NBUF]
    valss = ring[NBUF:2 * NBUF]
    cats = ring[2 * NBUF:3 * NBUF]
    cpos = ring[3 * NBUF:4 * NBUF]
    vpos = ring[4 * NBUF:5 * NBUF]
    semgs = ring[5 * NBUF:6 * NBUF]
    semos = ring[6 * NBUF:7 * NBUF]

    wid = lax.axis_index("s") * NC + lax.axis_index("c")
    wbase = wid * SEQ_PER_W * K

    pltpu.sync_copy(pos_hbm, pos_v)
    pltpu.sync_copy(ci_hbm.at[pl.ds(wbase, SEQ_PER_W * K)], ci_all)

    def start(c, b):
        goff = wbase + c * ROWS
        idx = ci_all.at[pl.ds(c * ROWS, ROWS)]
        pltpu.async_copy(
            bp_hbm.at[pl.ds(wid * SEQ_PER_W // S + c * CH // S, 1)],
            bps[b], semgs[b])
        pltpu.async_copy(vals_hbm.at[pl.ds(goff, ROWS)],
                         valss[b].at[pl.ds(0, ROWS)], semgs[b])
        pltpu.async_copy(wcat_hbm.at[idx], cats[b], semgs[b])

    def finish(c, b, j):
        bp, vals, cat = bps[b], valss[b], cats[b]
        cpo, vpo, semg, semo = cpos[b], vpos[b], semgs[b], semos[b]
        b_row = wid * SEQ_PER_W // S + c * CH // S
        s0 = (c % (S // CH)) * CH
        pltpu.make_async_copy(bp_hbm.at[pl.ds(0, 1)], bp, semg).wait()
        pltpu.make_async_copy(vals_hbm.at[pl.ds(0, ROWS)],
                              vals.at[pl.ds(0, ROWS)], semg).wait()
        pltpu.make_async_copy(wcat_hbm.at[pl.ds(0, ROWS)], cat, semg).wait()

        @pl.when(j >= 1)
        def _():
            pltpu.make_async_copy(cpo, cp_hbm.at[0, pl.ds(0, CH)], semo).wait()
            pltpu.make_async_copy(vpo, vp_hbm.at[0, pl.ds(0, CH)], semo).wait()

        for r in range(ROWS):
            g = vals[pl.ds((r // 16) * 16, 16)]
            valx[r, :] = lax.broadcast_in_dim(
                lax.slice_in_dim(g, r % 16, r % 16 + 1), (16,), (0,))

        for s_local in range(CH):
            def dv_body(dv, _):
                sl = pl.ds(dv * 16, 16)
                acc = bp[0, sl]
                for k in range(K):
                    r = s_local * K + k
                    cp = acc + pos_v[k, sl]
                    cpo[s_local, k, sl] = cp
                    vpo[s_local, k, sl] = cp + cat[r, pl.ds(dv * 16 + 2 * D, 16)]
                    acc = (acc + cat[r, sl]
                           + valx[r, :] * cat[r, pl.ds(dv * 16 + D, 16)])
                return ()

            lax.fori_loop(0, NV, dv_body, (), unroll=False)

        pltpu.async_copy(cpo, cp_hbm.at[b_row, pl.ds(s0, CH)], semo)
        pltpu.async_copy(vpo, vp_hbm.at[b_row, pl.ds(s0, CH)], semo)

    for b in range(NBUF):
        start(b, b)

    def body(j, _):
        for b in range(NBUF):
            c = j * NBUF + b
            finish(c, b, j)

            @pl.when(c + NBUF < NCHUNK)
            def _():
                start(c + NBUF, b)
        return ()

    lax.fori_loop(0, NCHUNK // NBUF, body, (), unroll=False)

    for b in range(NBUF):
        pltpu.make_async_copy(cpos[b], cp_hbm.at[0, pl.ds(0, CH)],
                              semos[b]).wait()
        pltpu.make_async_copy(vpos[b], vp_hbm.at[0, pl.ds(0, CH)],
                              semos[b]).wait()


@jax.jit
def _predictor_sc(ci_flat, vals_flat, base_predictor, wcat, pos_s):
    mesh = plsc.VectorSubcoreMesh(core_axis_name="c", subcore_axis_name="s")
    scratch = [
        pltpu.VMEM((K, D), jnp.float32),
        pltpu.VMEM((SEQ_PER_W * K,), jnp.int32),
        pltpu.VMEM((ROWS, 16), jnp.float32),
    ]
    scratch += [pltpu.VMEM((1, D), jnp.float32) for _ in range(NBUF)]
    scratch += [pltpu.VMEM((VPAD,), jnp.float32) for _ in range(NBUF)]
    scratch += [pltpu.VMEM((ROWS, 3 * D), jnp.float32) for _ in range(NBUF)]
    scratch += [pltpu.VMEM((CH, K, D), jnp.float32) for _ in range(NBUF)]
    scratch += [pltpu.VMEM((CH, K, D), jnp.float32) for _ in range(NBUF)]
    scratch += [pltpu.SemaphoreType.DMA for _ in range(NBUF)]
    scratch += [pltpu.SemaphoreType.DMA for _ in range(NBUF)]
    f = pl.kernel(
        _tec_body,
        out_type=(
            jax.ShapeDtypeStruct((B, S, K, D), jnp.float32),
            jax.ShapeDtypeStruct((B, S, K, D), jnp.float32),
        ),
        mesh=mesh,
        scratch_types=scratch,
    )
    return f(ci_flat, vals_flat, base_predictor, wcat, pos_s)


def kernel(class_indexes, value_indexes, base_predictor, W_present, W_value,
           W_query, position_embed):
    ci_flat = class_indexes.reshape(N * K)
    vals_flat = (value_indexes.astype(jnp.float32)
                 * (1.0 / (LEVELS - 1))).reshape(N * K)
    wcat = jnp.concatenate(
        [W_present, W_value, W_present + W_query], axis=1) * EMBED_SCALE
    pos_s = position_embed * EMBED_SCALE
    return _predictor_sc(ci_flat, vals_flat, base_predictor, wcat, pos_s)
